# baseline (speedup 1.0000x reference)
import functools

import jax
import jax.numpy as jnp
from jax import lax
from jax.experimental import pallas as pl
from jax.experimental.pallas import tpu as pltpu
from jax.experimental.pallas import tpu_sc as plsc

N = 10000
E = 160000
D = 256
H = 128
NC = 2
NS = 16
CH = 96
NB = 4
CPS = 108
E_PAD = NS * CH * CPS
NCH = E_PAD // CH
ACC_ROWS = N + 8
OR_STEP = 624
OR_LEN = 640
ZCH = 80


def _sc_body(xh, idx, zrows, out, i0, i1, i2, i3, rows, acc, gsem, ssem, dsem):
    c = lax.axis_index("c")
    s = lax.axis_index("s")
    ibuf = (i0, i1, i2, i3)
    for k in range(OR_LEN // ZCH):
        pltpu.sync_copy(zrows, acc.at[pl.ds(s * OR_STEP + k * ZCH, ZCH)])

    def dstage(j, b):
        pltpu.async_copy(idx.at[c, s * CPS + j], ibuf[b], dsem.at[b])

    def dwait(j, b):
        pltpu.make_async_copy(idx.at[c, s * CPS + j], ibuf[b], dsem.at[b]).wait()

    def gather_start(j, b):
        pltpu.async_copy(xh.at[ibuf[b].at[0]], rows.at[b], gsem.at[b])

    def gwait(j, b):
        pltpu.make_async_copy(xh.at[ibuf[b].at[0]], rows.at[b], gsem.at[b]).wait()

    def scatter_start(j, b):
        pltpu.async_copy(rows.at[b], acc.at[ibuf[b].at[1]], ssem.at[b], add=True)

    def swait(j, b):
        pltpu.make_async_copy(rows.at[b], acc.at[ibuf[b].at[1]], ssem.at[b]).wait()

    plsc.subcore_barrier()

    def step(j, b, first, do_dstage, do_gather):
        if not first:
            swait(j - 1, (b - 1) % NB)
        if do_dstage:
            dstage(j + 3, (b + 3) % NB)
        if do_gather:
            dwait(j + 2, (b + 2) % NB)
            gather_start(j + 2, (b + 2) % NB)
        gwait(j, b)
        scatter_start(j, b)

    dstage(0, 0)
    dstage(1, 1)
    dstage(2, 2)
    dwait(0, 0)
    gather_start(0, 0)
    dwait(1, 1)
    gather_start(1, 1)
    for b in range(NB):
        step(b, b, first=(b == 0), do_dstage=True, do_gather=True)

    def group_body(g, carry):
        for b in range(NB):
            step(g * NB + b, b, first=False, do_dstage=True, do_gather=True)
        return carry

    lax.fori_loop(1, CPS // NB - 1, group_body, 0)
    for b in range(NB):
        j = CPS - NB + b
        step(j, b, first=False, do_dstage=(j + 3 < CPS), do_gather=(j + 2 < CPS))
    swait(CPS - 1, NB - 1)

    plsc.subcore_barrier()
    pltpu.sync_copy(
        acc.at[pl.ds(s * OR_STEP, OR_LEN)], out.at[c, pl.ds(s * OR_STEP, OR_LEN)]
    )


_sc_scatter = functools.partial(
    pl.kernel,
    out_type=jax.ShapeDtypeStruct((NC, N, H), jnp.float32),
    mesh=plsc.VectorSubcoreMesh(core_axis_name="c", subcore_axis_name="s"),
    scratch_types=[
        pltpu.VMEM((2, CH), jnp.int32),
        pltpu.VMEM((2, CH), jnp.int32),
        pltpu.VMEM((2, CH), jnp.int32),
        pltpu.VMEM((2, CH), jnp.int32),
        pltpu.VMEM((NB, CH, H), jnp.float32),
        pltpu.VMEM_SHARED((ACC_ROWS, H), jnp.float32),
        pltpu.SemaphoreType.DMA((NB,)),
        pltpu.SemaphoreType.DMA((NB,)),
        pltpu.SemaphoreType.DMA((NB,)),
    ],
)(_sc_body)


BLK = 1000


def _tc_self_body(x_ref, w2t_ref, o_ref):
    o_ref[...] = jnp.dot(x_ref[...], w2t_ref[...], preferred_element_type=jnp.float32)


def _tc_self(x, w2t):
    return pl.pallas_call(
        _tc_self_body,
        grid=(N // BLK,),
        in_specs=[
            pl.BlockSpec((BLK, D), lambda i: (i, 0)),
            pl.BlockSpec((D, D), lambda i: (0, 0)),
        ],
        out_specs=pl.BlockSpec((BLK, D), lambda i: (i, 0)),
        out_shape=jax.ShapeDtypeStruct((N, D), jnp.float32),
    )(x, w2t)


def _tc_final_body(s_ref, a_ref, deg_ref, w1t_ref, o_ref):
    inv = 1.0 / deg_ref[...]
    o_ref[...] = (
        s_ref[...]
        + jnp.dot(a_ref[0] * inv, w1t_ref[0], preferred_element_type=jnp.float32)
        + jnp.dot(a_ref[1] * inv, w1t_ref[1], preferred_element_type=jnp.float32)
    )


def _tc_final(selfp, agg2, deg, w1t):
    return pl.pallas_call(
        _tc_final_body,
        grid=(N // BLK,),
        in_specs=[
            pl.BlockSpec((BLK, D), lambda i: (i, 0)),
            pl.BlockSpec((NC, BLK, H), lambda i: (0, i, 0)),
            pl.BlockSpec((BLK, 1), lambda i: (i, 0)),
            pl.BlockSpec((NC, H, D), lambda i: (0, 0, 0)),
        ],
        out_specs=pl.BlockSpec((BLK, D), lambda i: (i, 0)),
        out_shape=jax.ShapeDtypeStruct((N, D), jnp.float32),
    )(selfp, agg2, deg, w1t)


def kernel(x, edge_index, in_degree, W1, W2):
    src = edge_index[0]
    dst = edge_index[1]
    pad = E_PAD - E
    src_p = jnp.concatenate([jnp.int32(2) * src, jnp.zeros((pad,), jnp.int32)])
    dst_p = jnp.concatenate([dst, jnp.full((pad,), N, jnp.int32)])
    base = src_p.reshape(NCH, CH)
    srcs = jnp.stack([base, base + 1])
    dsts = jnp.broadcast_to(dst_p.reshape(1, NCH, CH), (NC, NCH, CH))
    idx = jnp.stack([srcs, dsts], axis=2)
    x2 = x.reshape(NC * N, H)
    zrows = jnp.zeros((ZCH, H), jnp.float32)
    agg2 = _sc_scatter(x2, idx, zrows)
    selfp = _tc_self(x, W2.T)
    w1t = W1.T.reshape(NC, H, D)
    out = _tc_final(selfp, agg2, in_degree.reshape(N, 1), w1t)
    return out


# speedup vs baseline: 1.1592x; 1.1592x over previous
import functools

import jax
import jax.numpy as jnp
from jax import lax
from jax.experimental import pallas as pl
from jax.experimental.pallas import tpu as pltpu
from jax.experimental.pallas import tpu_sc as plsc

N = 10000
E = 160000
D = 256
H = 128
NC = 2
NS = 16
CH = 32
NB = 4
CPS = 320
E_PAD = NS * CH * CPS
NCH = E_PAD // CH
ACC_ROWS = N + 8
OR_STEP = 624
OR_LEN = 640
ZCH = 80


def _sc_body(xh, idx, zrows, out, i0, i1, i2, i3, rows, acc, gsem, ssem, dsem):
    c = lax.axis_index("c")
    s = lax.axis_index("s")
    ibuf = (i0, i1, i2, i3)
    for k in range(OR_LEN // ZCH):
        pltpu.sync_copy(zrows, acc.at[pl.ds(s * OR_STEP + k * ZCH, ZCH)])

    def dstage(j, b):
        pltpu.async_copy(idx.at[c, s * CPS + j], ibuf[b], dsem.at[b])

    def dwait(j, b):
        pltpu.make_async_copy(idx.at[c, s * CPS + j], ibuf[b], dsem.at[b]).wait()

    def gather_start(j, b):
        pltpu.async_copy(xh.at[ibuf[b].at[0]], rows.at[b], gsem.at[b])

    def gwait(j, b):
        pltpu.make_async_copy(xh.at[ibuf[b].at[0]], rows.at[b], gsem.at[b]).wait()

    def scatter_start(j, b):
        pltpu.async_copy(rows.at[b], acc.at[ibuf[b].at[1]], ssem.at[b], add=True)

    def swait(j, b):
        pltpu.make_async_copy(rows.at[b], acc.at[ibuf[b].at[1]], ssem.at[b]).wait()

    plsc.subcore_barrier()

    def step(j, b, first, do_dstage, do_gather):
        if not first:
            swait(j - 1, (b - 1) % NB)
        if do_dstage:
            dstage(j + 3, (b + 3) % NB)
        if do_gather:
            dwait(j + 2, (b + 2) % NB)
            gather_start(j + 2, (b + 2) % NB)
        gwait(j, b)
        scatter_start(j, b)

    dstage(0, 0)
    dstage(1, 1)
    dstage(2, 2)
    dwait(0, 0)
    gather_start(0, 0)
    dwait(1, 1)
    gather_start(1, 1)
    for b in range(NB):
        step(b, b, first=(b == 0), do_dstage=True, do_gather=True)

    def group_body(g, carry):
        for b in range(NB):
            step(g * NB + b, b, first=False, do_dstage=True, do_gather=True)
        return carry

    lax.fori_loop(1, CPS // NB - 1, group_body, 0)
    for b in range(NB):
        j = CPS - NB + b
        step(j, b, first=False, do_dstage=(j + 3 < CPS), do_gather=(j + 2 < CPS))
    swait(CPS - 1, NB - 1)

    plsc.subcore_barrier()
    pltpu.sync_copy(
        acc.at[pl.ds(s * OR_STEP, OR_LEN)], out.at[c, pl.ds(s * OR_STEP, OR_LEN)]
    )


_sc_scatter = functools.partial(
    pl.kernel,
    out_type=jax.ShapeDtypeStruct((NC, N, H), jnp.float32),
    mesh=plsc.VectorSubcoreMesh(core_axis_name="c", subcore_axis_name="s"),
    scratch_types=[
        pltpu.VMEM((2, CH), jnp.int32),
        pltpu.VMEM((2, CH), jnp.int32),
        pltpu.VMEM((2, CH), jnp.int32),
        pltpu.VMEM((2, CH), jnp.int32),
        pltpu.VMEM((NB, CH, H), jnp.float32),
        pltpu.VMEM_SHARED((ACC_ROWS, H), jnp.float32),
        pltpu.SemaphoreType.DMA((NB,)),
        pltpu.SemaphoreType.DMA((NB,)),
        pltpu.SemaphoreType.DMA((NB,)),
    ],
)(_sc_body)


BLK = 1000


def _tc_self_body(x_ref, w2t_ref, o_ref):
    o_ref[...] = jnp.dot(x_ref[...], w2t_ref[...], preferred_element_type=jnp.float32)


def _tc_self(x, w2t):
    return pl.pallas_call(
        _tc_self_body,
        grid=(N // BLK,),
        in_specs=[
            pl.BlockSpec((BLK, D), lambda i: (i, 0)),
            pl.BlockSpec((D, D), lambda i: (0, 0)),
        ],
        out_specs=pl.BlockSpec((BLK, D), lambda i: (i, 0)),
        out_shape=jax.ShapeDtypeStruct((N, D), jnp.float32),
    )(x, w2t)


def _tc_final_body(s_ref, a_ref, deg_ref, w1t_ref, o_ref):
    inv = 1.0 / deg_ref[...]
    o_ref[...] = (
        s_ref[...]
        + jnp.dot(a_ref[0] * inv, w1t_ref[0], preferred_element_type=jnp.float32)
        + jnp.dot(a_ref[1] * inv, w1t_ref[1], preferred_element_type=jnp.float32)
    )


def _tc_final(selfp, agg2, deg, w1t):
    return pl.pallas_call(
        _tc_final_body,
        grid=(N // BLK,),
        in_specs=[
            pl.BlockSpec((BLK, D), lambda i: (i, 0)),
            pl.BlockSpec((NC, BLK, H), lambda i: (0, i, 0)),
            pl.BlockSpec((BLK, 1), lambda i: (i, 0)),
            pl.BlockSpec((NC, H, D), lambda i: (0, 0, 0)),
        ],
        out_specs=pl.BlockSpec((BLK, D), lambda i: (i, 0)),
        out_shape=jax.ShapeDtypeStruct((N, D), jnp.float32),
    )(selfp, agg2, deg, w1t)


def kernel(x, edge_index, in_degree, W1, W2):
    src = edge_index[0]
    dst = edge_index[1]
    pad = E_PAD - E
    src_p = jnp.concatenate([jnp.int32(2) * src, jnp.zeros((pad,), jnp.int32)])
    dst_p = jnp.concatenate([dst, jnp.full((pad,), N, jnp.int32)])
    base = src_p.reshape(NCH, CH)
    srcs = jnp.stack([base, base + 1])
    dsts = jnp.broadcast_to(dst_p.reshape(1, NCH, CH), (NC, NCH, CH))
    idx = jnp.stack([srcs, dsts], axis=2)
    x2 = x.reshape(NC * N, H)
    zrows = jnp.zeros((ZCH, H), jnp.float32)
    agg2 = _sc_scatter(x2, idx, zrows)
    selfp = _tc_self(x, W2.T)
    w1t = W1.T.reshape(NC, H, D)
    out = _tc_final(selfp, agg2, in_degree.reshape(N, 1), w1t)
    return out


# speedup vs baseline: 1.2781x; 1.1025x over previous
import functools

import jax
import jax.numpy as jnp
from jax import lax
from jax.experimental import pallas as pl
from jax.experimental.pallas import tpu as pltpu
from jax.experimental.pallas import tpu_sc as plsc

N = 10000
E = 160000
D = 256
H = 128
NC = 2
NS = 16
CH = 64
NB = 4
CPS = 160
E_PAD = NS * CH * CPS
NCH = E_PAD // CH
ACC_ROWS = N + 8
OR_STEP = 624
OR_LEN = 640
ZCH = 80


def _sc_body(xh, idx, zrows, out, i0, i1, i2, i3, rows, acc, gsem, ssem, dsem, zsem):
    c = lax.axis_index("c")
    s = lax.axis_index("s")
    ibuf = (i0, i1, i2, i3)
    for k in range(OR_LEN // ZCH):
        pltpu.async_copy(zrows, acc.at[pl.ds(s * OR_STEP + k * ZCH, ZCH)], zsem)
    for k in range(OR_LEN // ZCH):
        pltpu.make_async_copy(
            zrows, acc.at[pl.ds(s * OR_STEP + k * ZCH, ZCH)], zsem
        ).wait()

    def dstage(j, b):
        pltpu.async_copy(idx.at[c, s * CPS + j], ibuf[b], dsem.at[b])

    def dwait(j, b):
        pltpu.make_async_copy(idx.at[c, s * CPS + j], ibuf[b], dsem.at[b]).wait()

    def gather_start(j, b):
        pltpu.async_copy(xh.at[ibuf[b].at[0]], rows.at[b], gsem.at[b])

    def gwait(j, b):
        pltpu.make_async_copy(xh.at[ibuf[b].at[0]], rows.at[b], gsem.at[b]).wait()

    def scatter_start(j, b):
        pltpu.async_copy(rows.at[b], acc.at[ibuf[b].at[1]], ssem.at[b], add=True)

    def swait(j, b):
        pltpu.make_async_copy(rows.at[b], acc.at[ibuf[b].at[1]], ssem.at[b]).wait()

    plsc.subcore_barrier()

    def step(j, b, first, do_dstage, do_gather):
        if not first:
            swait(j - 1, (b - 1) % NB)
        if do_dstage:
            dstage(j + 3, (b + 3) % NB)
        if do_gather:
            dwait(j + 2, (b + 2) % NB)
            gather_start(j + 2, (b + 2) % NB)
        gwait(j, b)
        scatter_start(j, b)

    dstage(0, 0)
    dstage(1, 1)
    dstage(2, 2)
    dwait(0, 0)
    gather_start(0, 0)
    dwait(1, 1)
    gather_start(1, 1)
    for b in range(NB):
        step(b, b, first=(b == 0), do_dstage=True, do_gather=True)

    def group_body(g, carry):
        for b in range(NB):
            step(g * NB + b, b, first=False, do_dstage=True, do_gather=True)
        return carry

    lax.fori_loop(1, CPS // NB - 1, group_body, 0)
    for b in range(NB):
        j = CPS - NB + b
        step(j, b, first=False, do_dstage=(j + 3 < CPS), do_gather=(j + 2 < CPS))
    swait(CPS - 1, NB - 1)

    plsc.subcore_barrier()
    pltpu.sync_copy(
        acc.at[pl.ds(s * OR_STEP, OR_LEN)], out.at[c, pl.ds(s * OR_STEP, OR_LEN)]
    )


_sc_scatter = functools.partial(
    pl.kernel,
    out_type=jax.ShapeDtypeStruct((NC, N, H), jnp.float32),
    mesh=plsc.VectorSubcoreMesh(core_axis_name="c", subcore_axis_name="s"),
    scratch_types=[
        pltpu.VMEM((2, CH), jnp.int32),
        pltpu.VMEM((2, CH), jnp.int32),
        pltpu.VMEM((2, CH), jnp.int32),
        pltpu.VMEM((2, CH), jnp.int32),
        pltpu.VMEM((NB, CH, H), jnp.float32),
        pltpu.VMEM_SHARED((ACC_ROWS, H), jnp.float32),
        pltpu.SemaphoreType.DMA((NB,)),
        pltpu.SemaphoreType.DMA((NB,)),
        pltpu.SemaphoreType.DMA((NB,)),
        pltpu.SemaphoreType.DMA,
    ],
)(_sc_body)


BLK = 1000


def _tc_self_body(x_ref, w2t_ref, o_ref):
    o_ref[...] = jnp.dot(x_ref[...], w2t_ref[...], preferred_element_type=jnp.float32)


def _tc_self(x, w2t):
    return pl.pallas_call(
        _tc_self_body,
        grid=(N // BLK,),
        in_specs=[
            pl.BlockSpec((BLK, D), lambda i: (i, 0)),
            pl.BlockSpec((D, D), lambda i: (0, 0)),
        ],
        out_specs=pl.BlockSpec((BLK, D), lambda i: (i, 0)),
        out_shape=jax.ShapeDtypeStruct((N, D), jnp.float32),
    )(x, w2t)


def _tc_final_body(s_ref, a_ref, deg_ref, w1t_ref, o_ref):
    inv = 1.0 / deg_ref[...]
    o_ref[...] = (
        s_ref[...]
        + jnp.dot(a_ref[0] * inv, w1t_ref[0], preferred_element_type=jnp.float32)
        + jnp.dot(a_ref[1] * inv, w1t_ref[1], preferred_element_type=jnp.float32)
    )


def _tc_final(selfp, agg2, deg, w1t):
    return pl.pallas_call(
        _tc_final_body,
        grid=(N // BLK,),
        in_specs=[
            pl.BlockSpec((BLK, D), lambda i: (i, 0)),
            pl.BlockSpec((NC, BLK, H), lambda i: (0, i, 0)),
            pl.BlockSpec((BLK, 1), lambda i: (i, 0)),
            pl.BlockSpec((NC, H, D), lambda i: (0, 0, 0)),
        ],
        out_specs=pl.BlockSpec((BLK, D), lambda i: (i, 0)),
        out_shape=jax.ShapeDtypeStruct((N, D), jnp.float32),
    )(selfp, agg2, deg, w1t)


def kernel(x, edge_index, in_degree, W1, W2):
    src = edge_index[0]
    dst = edge_index[1]
    pad = E_PAD - E
    src_p = jnp.concatenate([jnp.int32(2) * src, jnp.zeros((pad,), jnp.int32)])
    dst_p = jnp.concatenate([dst, jnp.full((pad,), N, jnp.int32)])
    base = src_p.reshape(NCH, CH)
    srcs = jnp.stack([base, base + 1])
    dsts = jnp.broadcast_to(dst_p.reshape(1, NCH, CH), (NC, NCH, CH))
    idx = jnp.stack([srcs, dsts], axis=2)
    x2 = x.reshape(NC * N, H)
    zrows = jnp.zeros((ZCH, H), jnp.float32)
    agg2 = _sc_scatter(x2, idx, zrows)
    selfp = _tc_self(x, W2.T)
    w1t = W1.T.reshape(NC, H, D)
    out = _tc_final(selfp, agg2, in_degree.reshape(N, 1), w1t)
    return out


# speedup vs baseline: 1.2934x; 1.0119x over previous
import functools

import jax
import jax.numpy as jnp
from jax import lax
from jax.experimental import pallas as pl
from jax.experimental.pallas import tpu as pltpu
from jax.experimental.pallas import tpu_sc as plsc

N = 10000
E = 160000
D = 256
H = 128
NC = 2
NS = 16
CH = 64
NB = 4
CPS = 160
E_PAD = NS * CH * CPS
NCH = E_PAD // CH
ACC_ROWS = N + 8
OR_STEP = 624
OR_LEN = 640
ZCH = 80


def _sc_body(xh, idx, zrows, out, i0, i1, i2, i3, rows, acc, gsem, ssem, dsem, zsem):
    c = lax.axis_index("c")
    s = lax.axis_index("s")
    ibuf = (i0, i1, i2, i3)
    for k in range(OR_LEN // ZCH):
        pltpu.async_copy(zrows, acc.at[pl.ds(s * OR_STEP + k * ZCH, ZCH)], zsem)
    for k in range(OR_LEN // ZCH):
        pltpu.make_async_copy(
            zrows, acc.at[pl.ds(s * OR_STEP + k * ZCH, ZCH)], zsem
        ).wait()

    def dstage(j, b):
        pltpu.async_copy(idx.at[c, s * CPS + j], ibuf[b], dsem.at[b])

    def dwait(j, b):
        pltpu.make_async_copy(idx.at[c, s * CPS + j], ibuf[b], dsem.at[b]).wait()

    def gather_start(j, b):
        pltpu.async_copy(xh.at[ibuf[b].at[0]], rows.at[b], gsem.at[b])

    def gwait(j, b):
        pltpu.make_async_copy(xh.at[ibuf[b].at[0]], rows.at[b], gsem.at[b]).wait()

    def scatter_start(j, b):
        pltpu.async_copy(rows.at[b], acc.at[ibuf[b].at[1]], ssem.at[b], add=True)

    def swait(j, b):
        pltpu.make_async_copy(rows.at[b], acc.at[ibuf[b].at[1]], ssem.at[b]).wait()

    plsc.subcore_barrier()

    def step(j, b, first, do_dstage, do_gather):
        if not first:
            swait(j - 1, (b - 1) % NB)
        if do_dstage:
            dstage(j + 3, (b + 3) % NB)
        if do_gather:
            dwait(j + 2, (b + 2) % NB)
            gather_start(j + 2, (b + 2) % NB)
        gwait(j, b)
        scatter_start(j, b)

    dstage(0, 0)
    dstage(1, 1)
    dstage(2, 2)
    dwait(0, 0)
    gather_start(0, 0)
    dwait(1, 1)
    gather_start(1, 1)
    for b in range(NB):
        step(b, b, first=(b == 0), do_dstage=True, do_gather=True)

    def group_body(g, carry):
        for b in range(NB):
            step(g * NB + b, b, first=False, do_dstage=True, do_gather=True)
        return carry

    lax.fori_loop(1, CPS // NB - 1, group_body, 0)
    for b in range(NB):
        j = CPS - NB + b
        step(j, b, first=False, do_dstage=(j + 3 < CPS), do_gather=(j + 2 < CPS))
    swait(CPS - 1, NB - 1)

    plsc.subcore_barrier()
    pltpu.sync_copy(
        acc.at[pl.ds(s * OR_STEP, OR_LEN)], out.at[c, pl.ds(s * OR_STEP, OR_LEN)]
    )


_sc_scatter = functools.partial(
    pl.kernel,
    out_type=jax.ShapeDtypeStruct((NC, N, H), jnp.float32),
    mesh=plsc.VectorSubcoreMesh(core_axis_name="c", subcore_axis_name="s"),
    scratch_types=[
        pltpu.VMEM((2, CH), jnp.int32),
        pltpu.VMEM((2, CH), jnp.int32),
        pltpu.VMEM((2, CH), jnp.int32),
        pltpu.VMEM((2, CH), jnp.int32),
        pltpu.VMEM((NB, CH, H), jnp.float32),
        pltpu.VMEM_SHARED((ACC_ROWS, H), jnp.float32),
        pltpu.SemaphoreType.DMA((NB,)),
        pltpu.SemaphoreType.DMA((NB,)),
        pltpu.SemaphoreType.DMA((NB,)),
        pltpu.SemaphoreType.DMA,
    ],
)(_sc_body)


BLK = 1000


def _tc_final_body(x_ref, a_ref, deg_ref, w1t_ref, w2t_ref, o_ref):
    inv = 1.0 / deg_ref[...]
    o_ref[...] = (
        jnp.dot(x_ref[...], w2t_ref[...], preferred_element_type=jnp.float32)
        + jnp.dot(a_ref[0] * inv, w1t_ref[0], preferred_element_type=jnp.float32)
        + jnp.dot(a_ref[1] * inv, w1t_ref[1], preferred_element_type=jnp.float32)
    )


def _tc_final(x, agg2, deg, w1t, w2t):
    return pl.pallas_call(
        _tc_final_body,
        grid=(N // BLK,),
        in_specs=[
            pl.BlockSpec((BLK, D), lambda i: (i, 0)),
            pl.BlockSpec((NC, BLK, H), lambda i: (0, i, 0)),
            pl.BlockSpec((BLK, 1), lambda i: (i, 0)),
            pl.BlockSpec((NC, H, D), lambda i: (0, 0, 0)),
            pl.BlockSpec((D, D), lambda i: (0, 0)),
        ],
        out_specs=pl.BlockSpec((BLK, D), lambda i: (i, 0)),
        out_shape=jax.ShapeDtypeStruct((N, D), jnp.float32),
    )(x, agg2, deg, w1t, w2t)


def kernel(x, edge_index, in_degree, W1, W2):
    src = edge_index[0]
    dst = edge_index[1]
    pad = E_PAD - E
    src_p = jnp.concatenate([jnp.int32(2) * src, jnp.zeros((pad,), jnp.int32)])
    dst_p = jnp.concatenate([dst, jnp.full((pad,), N, jnp.int32)])
    base = src_p.reshape(NCH, CH)
    srcs = jnp.stack([base, base + 1])
    dsts = jnp.broadcast_to(dst_p.reshape(1, NCH, CH), (NC, NCH, CH))
    idx = jnp.stack([srcs, dsts], axis=2)
    x2 = x.reshape(NC * N, H)
    zrows = jnp.zeros((ZCH, H), jnp.float32)
    agg2 = _sc_scatter(x2, idx, zrows)
    w1t = W1.T.reshape(NC, H, D)
    out = _tc_final(x, agg2, in_degree.reshape(N, 1), w1t, W2.T)
    return out
